# baseline (speedup 1.0000x reference)
import jax
import jax.numpy as jnp
from jax import lax
from jax.experimental import pallas as pl
from jax.experimental.pallas import tpu as pltpu
from jax.experimental.pallas import tpu_sc as plsc

_B = 16
_NC = 2
_NS = 16
_NW = _NC * _NS


def _sc_body(states_hbm, out_hbm, rows_v, sem_in, sem_out):
    rows_per_w = rows_v.shape[0]
    half = rows_per_w // 2
    wid = lax.axis_index("c") * _NS + lax.axis_index("s")
    base = wid * rows_per_w
    stages = [
        pltpu.make_async_copy(
            states_hbm.at[0, pl.ds(base + h * half, half)],
            rows_v.at[pl.ds(h * half, half)],
            sem_in,
        )
        for h in range(2)
    ]
    for st in stages:
        st.start()
    writes = []
    for h in range(2):
        stages[h].wait()
        for j in range(_B):
            b = lax.rem(wid + j, _B)
            c = pltpu.make_async_copy(
                rows_v.at[pl.ds(h * half, half)],
                out_hbm.at[b, pl.ds(base + h * half, half)],
                sem_out,
            )
            c.start()
            writes.append(c)
    for c in writes:
        c.wait()


def kernel(states, batch_size):
    del batch_size
    _, S, D = states.shape
    rows_per_w = S // _NW
    sc_call = pl.kernel(
        _sc_body,
        out_type=jax.ShapeDtypeStruct((_B, S, D), states.dtype),
        mesh=plsc.VectorSubcoreMesh(core_axis_name="c", subcore_axis_name="s"),
        scratch_types=[
            pltpu.MemorySpace.VMEM((rows_per_w, D), states.dtype),
            pltpu.SemaphoreType.DMA,
            pltpu.SemaphoreType.DMA,
        ],
    )
    return sc_call(states)


# speedup vs baseline: 1.0039x; 1.0039x over previous
import jax
import jax.numpy as jnp
from jax import lax
from jax.experimental import pallas as pl
from jax.experimental.pallas import tpu as pltpu
from jax.experimental.pallas import tpu_sc as plsc

_B = 16
_NC = 2
_NS = 16
_NW = _NC * _NS


def _sc_body(states_hbm, out_hbm, rows_v, sem_in, sem_out):
    rows_per_w = rows_v.shape[0]
    half = rows_per_w // 2
    wid = lax.axis_index("c") * _NS + lax.axis_index("s")
    base = wid * rows_per_w
    stages = [
        pltpu.make_async_copy(
            states_hbm.at[0, pl.ds(base + h * half, half)],
            rows_v.at[pl.ds(h * half, half)],
            sem_in,
        )
        for h in range(2)
    ]
    for st in stages:
        st.start()
    writes = []
    for h in range(2):
        stages[h].wait()
        for b in range(_B):
            c = pltpu.make_async_copy(
                rows_v.at[pl.ds(h * half, half)],
                out_hbm.at[b, pl.ds(base + h * half, half)],
                sem_out,
            )
            c.start()
            writes.append(c)
    for c in writes:
        c.wait()


def kernel(states, batch_size):
    del batch_size
    _, S, D = states.shape
    rows_per_w = S // _NW
    sc_call = pl.kernel(
        _sc_body,
        out_type=jax.ShapeDtypeStruct((_B, S, D), states.dtype),
        mesh=plsc.VectorSubcoreMesh(core_axis_name="c", subcore_axis_name="s"),
        scratch_types=[
            pltpu.MemorySpace.VMEM((rows_per_w, D), states.dtype),
            pltpu.SemaphoreType.DMA,
            pltpu.SemaphoreType.DMA,
        ],
    )
    return sc_call(states)
